# baseline (device time: 64810 ns/iter reference)
import jax
import jax.numpy as jnp
from jax import lax
from jax.experimental import pallas as pl
from jax.experimental.pallas import tpu as pltpu

N_DEV = 8
M = 1024
N = 1024
CHUNK = M // N_DEV


def kernel(A, B):
    def body(a_ref, b_ref, out_ref, p_ref, send_buf, recv_buf, send_sems, recv_sems):
        me = lax.axis_index("i")
        right = lax.rem(me + 1, N_DEV)

        p_ref[...] = jnp.dot(
            a_ref[...], b_ref[...], preferred_element_type=jnp.float32
        )

        for h in range(N_DEV - 1):
            c = lax.rem(me - 1 - h + 2 * N_DEV, N_DEV)
            row = c * CHUNK
            val = p_ref[pl.ds(row, CHUNK), :]
            if h > 0:
                val = val + recv_buf[h - 1]
            send_buf[h] = val
            rdma = pltpu.make_async_remote_copy(
                src_ref=send_buf.at[h],
                dst_ref=recv_buf.at[h],
                send_sem=send_sems.at[h],
                recv_sem=recv_sems.at[h],
                device_id=(right,),
                device_id_type=pl.DeviceIdType.MESH,
            )
            rdma.start()
            rdma.wait()

        out_ref[...] = recv_buf[N_DEV - 2] + p_ref[pl.ds(me * CHUNK, CHUNK), :]

    return pl.pallas_call(
        body,
        out_shape=jax.ShapeDtypeStruct((CHUNK, N), jnp.float32),
        in_specs=[
            pl.BlockSpec(memory_space=pltpu.VMEM),
            pl.BlockSpec(memory_space=pltpu.VMEM),
        ],
        out_specs=pl.BlockSpec(memory_space=pltpu.VMEM),
        scratch_shapes=[
            pltpu.VMEM((M, N), jnp.float32),
            pltpu.VMEM((N_DEV - 1, CHUNK, N), jnp.float32),
            pltpu.VMEM((N_DEV - 1, CHUNK, N), jnp.float32),
            pltpu.SemaphoreType.DMA((N_DEV - 1,)),
            pltpu.SemaphoreType.DMA((N_DEV - 1,)),
        ],
    )(A, B)


# device time: 44712 ns/iter; 1.4495x vs baseline; 1.4495x over previous
import jax
import jax.numpy as jnp
from jax import lax
from jax.experimental import pallas as pl
from jax.experimental.pallas import tpu as pltpu

N_DEV = 8
M = 1024
N = 1024
CHUNK = M // N_DEV


def kernel(A, B):
    def body(a_ref, b_ref, out_ref, send_buf, recv_buf, send_sems, recv_sems):
        me = lax.axis_index("i")

        sends = []
        for k in range(1, N_DEV):
            dest = lax.rem(me + k, N_DEV)
            send_buf[k - 1] = jnp.dot(
                a_ref[pl.ds(dest * CHUNK, CHUNK), :],
                b_ref[...],
                preferred_element_type=jnp.float32,
            )
            rdma = pltpu.make_async_remote_copy(
                src_ref=send_buf.at[k - 1],
                dst_ref=recv_buf.at[me],
                send_sem=send_sems.at[k - 1],
                recv_sem=recv_sems.at[me],
                device_id=(dest,),
                device_id_type=pl.DeviceIdType.MESH,
            )
            rdma.start()
            sends.append(rdma)

        recv_buf[pl.ds(me, 1)] = jnp.dot(
            a_ref[pl.ds(me * CHUNK, CHUNK), :],
            b_ref[...],
            preferred_element_type=jnp.float32,
        )[None]

        for k in range(1, N_DEV):
            src = lax.rem(me + k, N_DEV)
            recv = pltpu.make_async_remote_copy(
                src_ref=send_buf.at[0],
                dst_ref=recv_buf.at[src],
                send_sem=send_sems.at[0],
                recv_sem=recv_sems.at[src],
                device_id=(me,),
                device_id_type=pl.DeviceIdType.MESH,
            )
            recv.wait_recv()

        acc = recv_buf[0]
        for j in range(1, N_DEV):
            acc = acc + recv_buf[j]
        out_ref[...] = acc

        for rdma in sends:
            rdma.wait_send()

    return pl.pallas_call(
        body,
        out_shape=jax.ShapeDtypeStruct((CHUNK, N), jnp.float32),
        in_specs=[
            pl.BlockSpec(memory_space=pltpu.VMEM),
            pl.BlockSpec(memory_space=pltpu.VMEM),
        ],
        out_specs=pl.BlockSpec(memory_space=pltpu.VMEM),
        scratch_shapes=[
            pltpu.VMEM((N_DEV - 1, CHUNK, N), jnp.float32),
            pltpu.VMEM((N_DEV, CHUNK, N), jnp.float32),
            pltpu.SemaphoreType.DMA((N_DEV - 1,)),
            pltpu.SemaphoreType.DMA((N_DEV,)),
        ],
    )(A, B)
